# baseline (speedup 1.0000x reference)
import jax
import jax.numpy as jnp
from jax import lax
from jax.experimental import pallas as pl
from jax.experimental.pallas import tpu as pltpu, tpu_sc as plsc

N = 10000
NP = 10048
C = 128
H = 8
DH = 16
E = 200000
B = 64
CHUNKS = 204
TILE_E = B * CHUNKS
EP = TILE_E * 16
STRIPE = NP // 16


def _edge_pass(sid, A, Bd, X, src, dst, cch, acc_out, ssum_out,
               src_v, dst_v, asrc_v, adst_v, ex_v, xr_v, cc_v,
               acc_sh, ssum_sh, sem_i, sem_g, sem_s):
    pltpu.sync_copy(cch, cc_v)
    cc = cc_v[...]
    base = sid * CHUNKS

    def issue_idx(j, ki):
        pltpu.async_copy(src.at[base + j], src_v[ki], sem_i[ki])
        pltpu.async_copy(dst.at[base + j], dst_v[ki], sem_i[ki])

    def wait_idx(j, ki):
        pltpu.make_async_copy(src.at[base + j], src_v[ki], sem_i[ki]).wait()
        pltpu.make_async_copy(dst.at[base + j], dst_v[ki], sem_i[ki]).wait()

    def issue_gathers(kd, ki):
        pltpu.async_copy(A.at[src_v[ki]], asrc_v[kd], sem_g[kd])
        pltpu.async_copy(Bd.at[dst_v[ki]], adst_v[kd], sem_g[kd])
        pltpu.async_copy(X.at[src_v[ki]], xr_v[kd], sem_g[kd])

    def wait_gathers(kd, ki):
        pltpu.make_async_copy(A.at[dst_v[ki]], asrc_v[kd], sem_g[kd]).wait()
        pltpu.make_async_copy(Bd.at[dst_v[ki]], adst_v[kd], sem_g[kd]).wait()
        pltpu.make_async_copy(X.at[dst_v[ki]], xr_v[kd], sem_g[kd]).wait()

    def compute(kd):
        @plsc.parallel_loop(0, B, unroll=4)
        def edge_body(b):
            v = asrc_v[kd][b, :] + adst_v[kd][b, :]
            v = jnp.where(v > 0.0, v, v * 0.2)
            e = jnp.exp(v - cc)
            ex_v[kd][b, :] = e
            for h in range(H):
                s = e[h]
                xr_v[kd][b, pl.ds(h * DH, DH)] = xr_v[kd][b, pl.ds(h * DH, DH)] * s

    def issue_scatters(kd, ki):
        pltpu.async_copy(xr_v[kd], acc_sh.at[dst_v[ki]], sem_s[kd], add=True)
        pltpu.async_copy(ex_v[kd], ssum_sh.at[dst_v[ki]], sem_s[kd], add=True)

    def drain_scatters(kd, ki):
        pltpu.make_async_copy(xr_v[kd], acc_sh.at[dst_v[ki]], sem_s[kd]).wait()
        pltpu.make_async_copy(ex_v[kd], ssum_sh.at[dst_v[ki]], sem_s[kd]).wait()

    issue_idx(0, 0)
    issue_idx(1, 1)
    wait_idx(0, 0)
    issue_gathers(0, 0)

    def superstep(js, carry):
        for kk in range(12):
            j = js * 12 + kk
            kd = kk % 3
            kdn = (kk + 1) % 3
            ki = kk % 4
            kin = (kk + 1) % 4
            ki2 = (kk + 2) % 4

            @pl.when(j >= 2)
            def _():
                drain_scatters(kdn, ki2)

            @pl.when(j + 2 < CHUNKS)
            def _():
                issue_idx(j + 2, ki2)

            @pl.when(j + 1 < CHUNKS)
            def _():
                wait_idx(j + 1, kin)
                issue_gathers(kdn, kin)

            wait_gathers(kd, ki)
            compute(kd)
            issue_scatters(kd, ki)
        return carry

    lax.fori_loop(0, CHUNKS // 12, superstep, 0)
    drain_scatters((CHUNKS - 2) % 3, (CHUNKS - 2) % 4)
    drain_scatters((CHUNKS - 1) % 3, (CHUNKS - 1) % 4)

    plsc.subcore_barrier()
    rows = pl.ds(sid * STRIPE, STRIPE)
    pltpu.sync_copy(acc_sh.at[rows], acc_out.at[rows])
    pltpu.sync_copy(ssum_sh.at[rows], ssum_out.at[rows])


def _sc_body(A_eo, Bd_eo, X_eo, src_eo, dst_eo, cc_eo,
             A_oo, Bd_oo, X_oo, src_oo, dst_oo, cc_oo,
             acc_eo, ssum_eo, acc_oo, ssum_oo,
             src_v, dst_v, asrc_v, adst_v, ex_v, xr_v, cc_v,
             acc_sh, ssum_sh, sem_i, sem_g, sem_s):
    cid = lax.axis_index("c")
    sid = lax.axis_index("s")

    z = jnp.zeros((16,), jnp.float32)

    @plsc.parallel_loop(0, B, unroll=4)
    def zero_body(b):
        for h in range(H):
            xr_v[0][b, pl.ds(h * DH, DH)] = z
        ex_v[0][b, :] = z
    full, rem = divmod(STRIPE, B)
    for k in range(full):
        rows = pl.ds(sid * STRIPE + k * B, B)
        pltpu.sync_copy(xr_v[0], acc_sh.at[rows])
        pltpu.sync_copy(ex_v[0], ssum_sh.at[rows])
    if rem:
        rows = pl.ds(sid * STRIPE + full * B, rem)
        pltpu.sync_copy(xr_v[0].at[pl.ds(0, rem)], acc_sh.at[rows])
        pltpu.sync_copy(ex_v[0].at[pl.ds(0, rem)], ssum_sh.at[rows])
    plsc.subcore_barrier()

    @pl.when(cid == 0)
    def _():
        _edge_pass(sid, A_eo, Bd_eo, X_eo, src_eo, dst_eo, cc_eo,
                   acc_eo, ssum_eo,
                   src_v, dst_v, asrc_v, adst_v, ex_v, xr_v, cc_v,
                   acc_sh, ssum_sh, sem_i, sem_g, sem_s)

    @pl.when(cid == 1)
    def _():
        _edge_pass(sid, A_oo, Bd_oo, X_oo, src_oo, dst_oo, cc_oo,
                   acc_oo, ssum_oo,
                   src_v, dst_v, asrc_v, adst_v, ex_v, xr_v, cc_v,
                   acc_sh, ssum_sh, sem_i, sem_g, sem_s)


@jax.jit
def _sc_call(A_eo, Bd_eo, X_eo, src_eo, dst_eo, cc_eo,
             A_oo, Bd_oo, X_oo, src_oo, dst_oo, cc_oo):
    mesh = plsc.VectorSubcoreMesh(core_axis_name="c", subcore_axis_name="s")
    f = pl.kernel(
        _sc_body,
        mesh=mesh,
        compiler_params=pltpu.CompilerParams(use_tc_tiling_on_sc=False),
        out_type=(
            jax.ShapeDtypeStruct((NP, C), jnp.float32),
            jax.ShapeDtypeStruct((NP, 16), jnp.float32),
            jax.ShapeDtypeStruct((NP, C), jnp.float32),
            jax.ShapeDtypeStruct((NP, 16), jnp.float32),
        ),
        scratch_types=[
            [pltpu.VMEM((B,), jnp.int32)] * 4,
            [pltpu.VMEM((B,), jnp.int32)] * 4,
            [pltpu.VMEM((B, 16), jnp.float32)] * 3,
            [pltpu.VMEM((B, 16), jnp.float32)] * 3,
            [pltpu.VMEM((B, 16), jnp.float32)] * 3,
            [pltpu.VMEM((B, C), jnp.float32)] * 3,
            pltpu.VMEM((16,), jnp.float32),
            pltpu.VMEM_SHARED((NP, C), jnp.float32),
            pltpu.VMEM_SHARED((NP, 16), jnp.float32),
            [pltpu.SemaphoreType.DMA] * 4,
            [pltpu.SemaphoreType.DMA] * 3,
            [pltpu.SemaphoreType.DMA] * 3,
        ],
    )
    return f(A_eo, Bd_eo, X_eo, src_eo, dst_eo, cc_eo,
             A_oo, Bd_oo, X_oo, src_oo, dst_oo, cc_oo)


def _prep_type(x_src, asrc, adst, ei):
    c = asrc.max(0) + adst.max(0)
    cc = jnp.concatenate([c, jnp.zeros((8,), jnp.float32)])
    A = jnp.zeros((N + 1, 16), jnp.float32)
    A = A.at[:N, :H].set(asrc)
    A = A.at[N, :].set(-1e30)
    Bd = jnp.zeros((N, 16), jnp.float32).at[:, :H].set(adst)
    X = jnp.concatenate([x_src, jnp.zeros((1, C), jnp.float32)], axis=0)
    src = jnp.full((EP,), N, jnp.int32).at[:E].set(ei[0].astype(jnp.int32))
    dst = jnp.zeros((EP,), jnp.int32).at[:E].set(ei[1].astype(jnp.int32))
    return A, Bd, X, src.reshape(EP // B, B), dst.reshape(EP // B, B), cc


def kernel(x_openie, x_entity, ei_oe, ei_eo, ei_oo, Wp_o, bp_o, Wp_e, bp_e,
           att_src_oe, att_dst_oe, att_src_eo, att_dst_eo, att_src_oo, att_dst_oo,
           k_lin_W, k_lin_b, q, lin_W, lin_b):
    x_o = x_openie @ Wp_o + bp_o
    x_e = x_entity @ Wp_e + bp_e

    def head_dot(x, att):
        return (x.reshape(N, H, DH) * att).sum(-1)

    asrc_eo = head_dot(x_e, att_src_eo)
    adst_eo = head_dot(x_o, att_dst_eo)
    asrc_oo = head_dot(x_o, att_src_oo)
    adst_oo = head_dot(x_o, att_dst_oo)

    args_eo = _prep_type(x_e, asrc_eo, adst_eo, ei_eo)
    args_oo = _prep_type(x_o, asrc_oo, adst_oo, ei_oo)

    acc_eo, ssum_eo, acc_oo, ssum_oo = _sc_call(*args_eo, *args_oo)

    def finish(acc, ssum):
        o = acc[:N].reshape(N, H, DH) / (ssum[:N, :H, None] + 1e-16)
        return jax.nn.relu(o).reshape(N, H * DH)

    out_eo = finish(acc_eo, ssum_eo)
    out_oo = finish(acc_oo, ssum_oo)

    out = jnp.stack([out_eo, out_oo])
    kk = jnp.tanh(out @ k_lin_W + k_lin_b).mean(axis=1)
    score = (q * kk).sum(-1)
    attn = jax.nn.softmax(score, axis=0)
    h_openie = (attn[:, None, None] * out).sum(0)
    pooled = h_openie.mean(axis=0, keepdims=True)
    return pooled @ lin_W + lin_b


# speedup vs baseline: 1.6492x; 1.6492x over previous
import jax
import jax.numpy as jnp
from jax import lax
from jax.experimental import pallas as pl
from jax.experimental.pallas import tpu as pltpu, tpu_sc as plsc

N = 10000
NP = 10048
C = 128
H = 8
DH = 16
E = 200000
B = 64
CHUNKS = 198
TILE_E = B * CHUNKS
EP = TILE_E * 16
STRIPE = NP // 16


def _edge_pass(sid, A, Bd, X, src, dst, cch, acc_out, ssum_out,
               src_v, dst_v, asrc_v, adst_v, ex_v, xr_v, cc_v,
               acc_sh, ssum_sh, sem_i, sem_g, sem_s):
    pltpu.sync_copy(cch, cc_v)
    cc = cc_v[...]
    base = sid * CHUNKS

    def issue_idx(j, k):
        pltpu.async_copy(src.at[base + j], src_v[k], sem_i[k])
        pltpu.async_copy(dst.at[base + j], dst_v[k], sem_i[k])

    def wait_idx(j, k):
        pltpu.make_async_copy(src.at[base + j], src_v[k], sem_i[k]).wait()
        pltpu.make_async_copy(dst.at[base + j], dst_v[k], sem_i[k]).wait()

    def issue_gathers(k):
        pltpu.async_copy(A.at[src_v[k]], asrc_v[k], sem_g[k])
        pltpu.async_copy(Bd.at[dst_v[k]], adst_v[k], sem_g[k])
        pltpu.async_copy(X.at[src_v[k]], xr_v[k], sem_g[k])

    def wait_gathers(k):
        pltpu.make_async_copy(A.at[dst_v[k]], asrc_v[k], sem_g[k]).wait()
        pltpu.make_async_copy(Bd.at[dst_v[k]], adst_v[k], sem_g[k]).wait()
        pltpu.make_async_copy(X.at[dst_v[k]], xr_v[k], sem_g[k]).wait()

    def compute(k):
        @plsc.parallel_loop(0, B, unroll=4)
        def edge_body(b):
            v = asrc_v[k][b, :] + adst_v[k][b, :]
            v = jnp.where(v > 0.0, v, v * 0.2)
            e = jnp.exp(v - cc)
            ex_v[k][b, :] = e
            for h in range(H):
                s = e[h]
                xr_v[k][b, pl.ds(h * DH, DH)] = xr_v[k][b, pl.ds(h * DH, DH)] * s

    def issue_scatters(k):
        pltpu.async_copy(xr_v[k], acc_sh.at[dst_v[k]], sem_s[k], add=True)
        pltpu.async_copy(ex_v[k], ssum_sh.at[dst_v[k]], sem_s[k], add=True)

    def drain_scatters(k):
        pltpu.make_async_copy(xr_v[k], acc_sh.at[dst_v[k]], sem_s[k]).wait()
        pltpu.make_async_copy(ex_v[k], ssum_sh.at[dst_v[k]], sem_s[k]).wait()

    issue_idx(0, 0)
    issue_idx(1, 1)
    wait_idx(0, 0)
    issue_gathers(0)

    def superstep(js, carry):
        for k in range(3):
            j = js * 3 + k
            kn = (k + 1) % 3
            k2 = (k + 2) % 3
            issue_idx(j + 2, k2)
            wait_idx(j + 1, kn)
            issue_gathers(kn)
            wait_gathers(k)
            compute(k)

            @pl.when(j >= 1)
            def _():
                drain_scatters(k2)
            issue_scatters(k)
        return carry

    lax.fori_loop(0, CHUNKS // 3 - 1, superstep, 0)
    for k in range(3):
        j = CHUNKS - 3 + k
        kn = (k + 1) % 3
        k2 = (k + 2) % 3
        if j + 2 < CHUNKS:
            issue_idx(j + 2, k2)
        if j + 1 < CHUNKS:
            wait_idx(j + 1, kn)
            issue_gathers(kn)
        wait_gathers(k)
        compute(k)
        drain_scatters(k2)
        issue_scatters(k)
    drain_scatters((CHUNKS - 1) % 3)

    plsc.subcore_barrier()
    rows = pl.ds(sid * STRIPE, STRIPE)
    pltpu.sync_copy(acc_sh.at[rows], acc_out.at[rows])
    pltpu.sync_copy(ssum_sh.at[rows], ssum_out.at[rows])


def _sc_body(A_eo, Bd_eo, X_eo, src_eo, dst_eo, cc_eo,
             A_oo, Bd_oo, X_oo, src_oo, dst_oo, cc_oo,
             acc_eo, ssum_eo, acc_oo, ssum_oo,
             src_v, dst_v, asrc_v, adst_v, ex_v, xr_v, cc_v,
             acc_sh, ssum_sh, sem_i, sem_g, sem_s):
    cid = lax.axis_index("c")
    sid = lax.axis_index("s")

    z = jnp.zeros((16,), jnp.float32)

    @plsc.parallel_loop(0, B, unroll=4)
    def zero_body(b):
        for h in range(H):
            xr_v[0][b, pl.ds(h * DH, DH)] = z
        ex_v[0][b, :] = z
    full, rem = divmod(STRIPE, B)
    for k in range(full):
        rows = pl.ds(sid * STRIPE + k * B, B)
        pltpu.sync_copy(xr_v[0], acc_sh.at[rows])
        pltpu.sync_copy(ex_v[0], ssum_sh.at[rows])
    if rem:
        rows = pl.ds(sid * STRIPE + full * B, rem)
        pltpu.sync_copy(xr_v[0].at[pl.ds(0, rem)], acc_sh.at[rows])
        pltpu.sync_copy(ex_v[0].at[pl.ds(0, rem)], ssum_sh.at[rows])
    plsc.subcore_barrier()

    @pl.when(cid == 0)
    def _():
        _edge_pass(sid, A_eo, Bd_eo, X_eo, src_eo, dst_eo, cc_eo,
                   acc_eo, ssum_eo,
                   src_v, dst_v, asrc_v, adst_v, ex_v, xr_v, cc_v,
                   acc_sh, ssum_sh, sem_i, sem_g, sem_s)

    @pl.when(cid == 1)
    def _():
        _edge_pass(sid, A_oo, Bd_oo, X_oo, src_oo, dst_oo, cc_oo,
                   acc_oo, ssum_oo,
                   src_v, dst_v, asrc_v, adst_v, ex_v, xr_v, cc_v,
                   acc_sh, ssum_sh, sem_i, sem_g, sem_s)


@jax.jit
def _sc_call(A_eo, Bd_eo, X_eo, src_eo, dst_eo, cc_eo,
             A_oo, Bd_oo, X_oo, src_oo, dst_oo, cc_oo):
    mesh = plsc.VectorSubcoreMesh(core_axis_name="c", subcore_axis_name="s")
    f = pl.kernel(
        _sc_body,
        mesh=mesh,
        compiler_params=pltpu.CompilerParams(use_tc_tiling_on_sc=False),
        out_type=(
            jax.ShapeDtypeStruct((NP, C), jnp.float32),
            jax.ShapeDtypeStruct((NP, 16), jnp.float32),
            jax.ShapeDtypeStruct((NP, C), jnp.float32),
            jax.ShapeDtypeStruct((NP, 16), jnp.float32),
        ),
        scratch_types=[
            [pltpu.VMEM((B,), jnp.int32)] * 3,
            [pltpu.VMEM((B,), jnp.int32)] * 3,
            [pltpu.VMEM((B, 16), jnp.float32)] * 3,
            [pltpu.VMEM((B, 16), jnp.float32)] * 3,
            [pltpu.VMEM((B, 16), jnp.float32)] * 3,
            [pltpu.VMEM((B, C), jnp.float32)] * 3,
            pltpu.VMEM((16,), jnp.float32),
            pltpu.VMEM_SHARED((NP, C), jnp.float32),
            pltpu.VMEM_SHARED((NP, 16), jnp.float32),
            [pltpu.SemaphoreType.DMA] * 3,
            [pltpu.SemaphoreType.DMA] * 3,
            [pltpu.SemaphoreType.DMA] * 3,
        ],
    )
    return f(A_eo, Bd_eo, X_eo, src_eo, dst_eo, cc_eo,
             A_oo, Bd_oo, X_oo, src_oo, dst_oo, cc_oo)


def _prep_type(x_src, asrc, adst, ei):
    c = asrc.max(0) + adst.max(0)
    cc = jnp.concatenate([c, jnp.zeros((8,), jnp.float32)])
    A = jnp.zeros((N + 1, 16), jnp.float32)
    A = A.at[:N, :H].set(asrc)
    A = A.at[N, :].set(-1e30)
    Bd = jnp.zeros((N, 16), jnp.float32).at[:, :H].set(adst)
    X = jnp.concatenate([x_src, jnp.zeros((1, C), jnp.float32)], axis=0)
    src = jnp.full((EP,), N, jnp.int32).at[:E].set(ei[0].astype(jnp.int32))
    dst = jnp.zeros((EP,), jnp.int32).at[:E].set(ei[1].astype(jnp.int32))
    return A, Bd, X, src.reshape(EP // B, B), dst.reshape(EP // B, B), cc


def kernel(x_openie, x_entity, ei_oe, ei_eo, ei_oo, Wp_o, bp_o, Wp_e, bp_e,
           att_src_oe, att_dst_oe, att_src_eo, att_dst_eo, att_src_oo, att_dst_oo,
           k_lin_W, k_lin_b, q, lin_W, lin_b):
    x_o = x_openie @ Wp_o + bp_o
    x_e = x_entity @ Wp_e + bp_e

    def head_dot(x, att):
        return (x.reshape(N, H, DH) * att).sum(-1)

    asrc_eo = head_dot(x_e, att_src_eo)
    adst_eo = head_dot(x_o, att_dst_eo)
    asrc_oo = head_dot(x_o, att_src_oo)
    adst_oo = head_dot(x_o, att_dst_oo)

    args_eo = _prep_type(x_e, asrc_eo, adst_eo, ei_eo)
    args_oo = _prep_type(x_o, asrc_oo, adst_oo, ei_oo)

    acc_eo, ssum_eo, acc_oo, ssum_oo = _sc_call(*args_eo, *args_oo)

    def finish(acc, ssum):
        o = acc[:N].reshape(N, H, DH) / (ssum[:N, :H, None] + 1e-16)
        return jax.nn.relu(o).reshape(N, H * DH)

    out_eo = finish(acc_eo, ssum_eo)
    out_oo = finish(acc_oo, ssum_oo)

    out = jnp.stack([out_eo, out_oo])
    kk = jnp.tanh(out @ k_lin_W + k_lin_b).mean(axis=1)
    score = (q * kk).sum(-1)
    attn = jax.nn.softmax(score, axis=0)
    h_openie = (attn[:, None, None] * out).sum(0)
    pooled = h_openie.mean(axis=0, keepdims=True)
    return pooled @ lin_W + lin_b


# speedup vs baseline: 1.7536x; 1.0633x over previous
import jax
import jax.numpy as jnp
from jax import lax
from jax.experimental import pallas as pl
from jax.experimental.pallas import tpu as pltpu, tpu_sc as plsc

N = 10000
NP = 10048
C = 128
H = 8
DH = 16
E = 200000
B = 64
CHUNKS = 198
TILE_E = B * CHUNKS
EP = TILE_E * 16
STRIPE = NP // 16


def _edge_pass(sid, A, Bd, X, src, dst, cch, acc_out,
               src_v, dst_v, asrc_v, adst_v, ex_v, xr_v, cc_v,
               acc_sh, ssum_sh, sem_i, sem_g, sem_s):
    pltpu.sync_copy(cch, cc_v)
    cc = cc_v[...]
    base = sid * CHUNKS

    def issue_idx(j, k):
        pltpu.async_copy(src.at[base + j], src_v[k], sem_i[k])
        pltpu.async_copy(dst.at[base + j], dst_v[k], sem_i[k])

    def wait_idx(j, k):
        pltpu.make_async_copy(src.at[base + j], src_v[k], sem_i[k]).wait()
        pltpu.make_async_copy(dst.at[base + j], dst_v[k], sem_i[k]).wait()

    def issue_gathers(k):
        pltpu.async_copy(A.at[src_v[k]], asrc_v[k], sem_g[k])
        pltpu.async_copy(Bd.at[dst_v[k]], adst_v[k], sem_g[k])
        pltpu.async_copy(X.at[src_v[k]], xr_v[k], sem_g[k])

    def wait_gathers(k):
        pltpu.make_async_copy(A.at[dst_v[k]], asrc_v[k], sem_g[k]).wait()
        pltpu.make_async_copy(Bd.at[dst_v[k]], adst_v[k], sem_g[k]).wait()
        pltpu.make_async_copy(X.at[dst_v[k]], xr_v[k], sem_g[k]).wait()

    def compute(k):
        @plsc.parallel_loop(0, B, unroll=4)
        def edge_body(b):
            v = asrc_v[k][b, :] + adst_v[k][b, :]
            v = jnp.where(v > 0.0, v, v * 0.2)
            e = jnp.exp(v - cc)
            ex_v[k][b, :] = e
            for h in range(H):
                s = e[h]
                xr_v[k][b, pl.ds(h * DH, DH)] = xr_v[k][b, pl.ds(h * DH, DH)] * s

    def issue_scatters(k):
        pltpu.async_copy(xr_v[k], acc_sh.at[dst_v[k]], sem_s[k], add=True)
        pltpu.async_copy(ex_v[k], ssum_sh.at[dst_v[k]], sem_s[k], add=True)

    def drain_scatters(k):
        pltpu.make_async_copy(xr_v[k], acc_sh.at[dst_v[k]], sem_s[k]).wait()
        pltpu.make_async_copy(ex_v[k], ssum_sh.at[dst_v[k]], sem_s[k]).wait()

    issue_idx(0, 0)
    issue_idx(1, 1)
    wait_idx(0, 0)
    issue_gathers(0)

    def superstep(js, carry):
        for k in range(3):
            j = js * 3 + k
            kn = (k + 1) % 3
            k2 = (k + 2) % 3
            issue_idx(j + 2, k2)
            wait_idx(j + 1, kn)
            issue_gathers(kn)
            wait_gathers(k)
            compute(k)

            @pl.when(j >= 1)
            def _():
                drain_scatters(k2)
            issue_scatters(k)
        return carry

    lax.fori_loop(0, CHUNKS // 3 - 1, superstep, 0)
    for k in range(3):
        j = CHUNKS - 3 + k
        kn = (k + 1) % 3
        k2 = (k + 2) % 3
        if j + 2 < CHUNKS:
            issue_idx(j + 2, k2)
        if j + 1 < CHUNKS:
            wait_idx(j + 1, kn)
            issue_gathers(kn)
        wait_gathers(k)
        compute(k)
        drain_scatters(k2)
        issue_scatters(k)
    drain_scatters((CHUNKS - 1) % 3)

    plsc.subcore_barrier()

    def finalize_block(row0, nrows):
        pltpu.sync_copy(acc_sh.at[pl.ds(row0, nrows)], xr_v[0].at[pl.ds(0, nrows)])
        pltpu.sync_copy(ssum_sh.at[pl.ds(row0, nrows)], ex_v[0].at[pl.ds(0, nrows)])

        @plsc.parallel_loop(0, nrows, unroll=4)
        def row_body(b):
            s_row = ex_v[0][b, :] + 1e-16
            for h in range(H):
                d = s_row[h]
                sl = pl.ds(h * DH, DH)
                xr_v[0][b, sl] = jnp.maximum(xr_v[0][b, sl] / d, 0.0)

        pltpu.sync_copy(xr_v[0].at[pl.ds(0, nrows)], acc_out.at[pl.ds(row0, nrows)])

    full, rem = divmod(STRIPE, B)
    for t in range(full):
        finalize_block(sid * STRIPE + t * B, B)
    if rem:
        finalize_block(sid * STRIPE + full * B, rem)


def _sc_body(A_eo, Bd_eo, X_eo, src_eo, dst_eo, cc_eo,
             A_oo, Bd_oo, X_oo, src_oo, dst_oo, cc_oo,
             acc_eo, acc_oo,
             src_v, dst_v, asrc_v, adst_v, ex_v, xr_v, cc_v,
             acc_sh, ssum_sh, sem_i, sem_g, sem_s):
    cid = lax.axis_index("c")
    sid = lax.axis_index("s")

    z = jnp.zeros((16,), jnp.float32)

    @plsc.parallel_loop(0, B, unroll=4)
    def zero_body(b):
        for h in range(H):
            xr_v[0][b, pl.ds(h * DH, DH)] = z
        ex_v[0][b, :] = z
    full, rem = divmod(STRIPE, B)
    for k in range(full):
        rows = pl.ds(sid * STRIPE + k * B, B)
        pltpu.sync_copy(xr_v[0], acc_sh.at[rows])
        pltpu.sync_copy(ex_v[0], ssum_sh.at[rows])
    if rem:
        rows = pl.ds(sid * STRIPE + full * B, rem)
        pltpu.sync_copy(xr_v[0].at[pl.ds(0, rem)], acc_sh.at[rows])
        pltpu.sync_copy(ex_v[0].at[pl.ds(0, rem)], ssum_sh.at[rows])
    plsc.subcore_barrier()

    @pl.when(cid == 0)
    def _():
        _edge_pass(sid, A_eo, Bd_eo, X_eo, src_eo, dst_eo, cc_eo,
                   acc_eo,
                   src_v, dst_v, asrc_v, adst_v, ex_v, xr_v, cc_v,
                   acc_sh, ssum_sh, sem_i, sem_g, sem_s)

    @pl.when(cid == 1)
    def _():
        _edge_pass(sid, A_oo, Bd_oo, X_oo, src_oo, dst_oo, cc_oo,
                   acc_oo,
                   src_v, dst_v, asrc_v, adst_v, ex_v, xr_v, cc_v,
                   acc_sh, ssum_sh, sem_i, sem_g, sem_s)


@jax.jit
def _sc_call(A_eo, Bd_eo, X_eo, src_eo, dst_eo, cc_eo,
             A_oo, Bd_oo, X_oo, src_oo, dst_oo, cc_oo):
    mesh = plsc.VectorSubcoreMesh(core_axis_name="c", subcore_axis_name="s")
    f = pl.kernel(
        _sc_body,
        mesh=mesh,
        compiler_params=pltpu.CompilerParams(use_tc_tiling_on_sc=False),
        out_type=(
            jax.ShapeDtypeStruct((NP, C), jnp.float32),
            jax.ShapeDtypeStruct((NP, C), jnp.float32),
        ),
        scratch_types=[
            [pltpu.VMEM((B,), jnp.int32)] * 3,
            [pltpu.VMEM((B,), jnp.int32)] * 3,
            [pltpu.VMEM((B, 16), jnp.float32)] * 3,
            [pltpu.VMEM((B, 16), jnp.float32)] * 3,
            [pltpu.VMEM((B, 16), jnp.float32)] * 3,
            [pltpu.VMEM((B, C), jnp.float32)] * 3,
            pltpu.VMEM((16,), jnp.float32),
            pltpu.VMEM_SHARED((NP, C), jnp.float32),
            pltpu.VMEM_SHARED((NP, 16), jnp.float32),
            [pltpu.SemaphoreType.DMA] * 3,
            [pltpu.SemaphoreType.DMA] * 3,
            [pltpu.SemaphoreType.DMA] * 3,
        ],
    )
    return f(A_eo, Bd_eo, X_eo, src_eo, dst_eo, cc_eo,
             A_oo, Bd_oo, X_oo, src_oo, dst_oo, cc_oo)


def _prep_type(x_src, asrc, adst, ei):
    c = asrc.max(0) + adst.max(0)
    cc = jnp.concatenate([c, jnp.zeros((8,), jnp.float32)])
    A = jnp.zeros((N + 1, 16), jnp.float32)
    A = A.at[:N, :H].set(asrc)
    A = A.at[N, :].set(-1e30)
    Bd = jnp.zeros((N, 16), jnp.float32).at[:, :H].set(adst)
    X = jnp.concatenate([x_src, jnp.zeros((1, C), jnp.float32)], axis=0)
    src = jnp.full((EP,), N, jnp.int32).at[:E].set(ei[0].astype(jnp.int32))
    dst = jnp.zeros((EP,), jnp.int32).at[:E].set(ei[1].astype(jnp.int32))
    return A, Bd, X, src.reshape(EP // B, B), dst.reshape(EP // B, B), cc


def kernel(x_openie, x_entity, ei_oe, ei_eo, ei_oo, Wp_o, bp_o, Wp_e, bp_e,
           att_src_oe, att_dst_oe, att_src_eo, att_dst_eo, att_src_oo, att_dst_oo,
           k_lin_W, k_lin_b, q, lin_W, lin_b):
    x_o = x_openie @ Wp_o + bp_o
    x_e = x_entity @ Wp_e + bp_e

    def head_dot(x, att):
        return (x.reshape(N, H, DH) * att).sum(-1)

    asrc_eo = head_dot(x_e, att_src_eo)
    adst_eo = head_dot(x_o, att_dst_eo)
    asrc_oo = head_dot(x_o, att_src_oo)
    adst_oo = head_dot(x_o, att_dst_oo)

    args_eo = _prep_type(x_e, asrc_eo, adst_eo, ei_eo)
    args_oo = _prep_type(x_o, asrc_oo, adst_oo, ei_oo)

    out_eo_p, out_oo_p = _sc_call(*args_eo, *args_oo)
    out_eo = out_eo_p[:N]
    out_oo = out_oo_p[:N]

    out = jnp.stack([out_eo, out_oo])
    kk = jnp.tanh(out @ k_lin_W + k_lin_b).mean(axis=1)
    score = (q * kk).sum(-1)
    attn = jax.nn.softmax(score, axis=0)
    h_openie = (attn[:, None, None] * out).sum(0)
    pooled = h_openie.mean(axis=0, keepdims=True)
    return pooled @ lin_W + lin_b


# speedup vs baseline: 1.7606x; 1.0040x over previous
import jax
import jax.numpy as jnp
from jax import lax
from jax.experimental import pallas as pl
from jax.experimental.pallas import tpu as pltpu, tpu_sc as plsc

N = 10000
NP = 10048
C = 128
H = 8
DH = 16
E = 200000
B = 64
CHUNKS = 198
TILE_E = B * CHUNKS
EP = TILE_E * 16
STRIPE = NP // 16


def _edge_pass(sid, A, Bd, X, src, dst, cch, acc_out,
               src_v, dst_v, asrc_v, adst_v, ex_v, xr_v, cc_v,
               acc_sh, ssum_sh, sem_i, sem_g, sem_s):
    pltpu.sync_copy(cch, cc_v)
    cc = cc_v[...]
    base = sid * CHUNKS

    def issue_idx(j, k):
        pltpu.async_copy(src.at[base + j], src_v[k], sem_i[k])
        pltpu.async_copy(dst.at[base + j], dst_v[k], sem_i[k])

    def wait_idx(j, k):
        pltpu.make_async_copy(src.at[base + j], src_v[k], sem_i[k]).wait()
        pltpu.make_async_copy(dst.at[base + j], dst_v[k], sem_i[k]).wait()

    def issue_gathers(k):
        pltpu.async_copy(A.at[src_v[k]], asrc_v[k], sem_g[k])
        pltpu.async_copy(Bd.at[dst_v[k]], adst_v[k], sem_g[k])
        pltpu.async_copy(X.at[src_v[k]], xr_v[k], sem_g[k])

    def wait_gathers(k):
        pltpu.make_async_copy(A.at[dst_v[k]], asrc_v[k], sem_g[k]).wait()
        pltpu.make_async_copy(Bd.at[dst_v[k]], adst_v[k], sem_g[k]).wait()
        pltpu.make_async_copy(X.at[dst_v[k]], xr_v[k], sem_g[k]).wait()

    def compute(k):
        @plsc.parallel_loop(0, B, unroll=4)
        def edge_body(b):
            v = asrc_v[k][b, :] + adst_v[k][b, :]
            v = jnp.where(v > 0.0, v, v * 0.2)
            e = jnp.exp(v - cc)
            ex_v[k][b, :] = e
            for h in range(H):
                s = e[h]
                xr_v[k][b, pl.ds(h * DH, DH)] = xr_v[k][b, pl.ds(h * DH, DH)] * s

    def issue_scatters(k):
        pltpu.async_copy(xr_v[k], acc_sh.at[dst_v[k]], sem_s[k], add=True)
        pltpu.async_copy(ex_v[k], ssum_sh.at[dst_v[k]], sem_s[k], add=True)

    def drain_scatters(k):
        pltpu.make_async_copy(xr_v[k], acc_sh.at[dst_v[k]], sem_s[k]).wait()
        pltpu.make_async_copy(ex_v[k], ssum_sh.at[dst_v[k]], sem_s[k]).wait()

    issue_idx(0, 0)
    issue_idx(1, 1)
    wait_idx(0, 0)
    issue_gathers(0)

    def superstep(js, carry):
        for k in range(3):
            j = js * 3 + k
            kn = (k + 1) % 3
            k2 = (k + 2) % 3
            issue_idx(j + 2, k2)
            wait_idx(j + 1, kn)
            issue_gathers(kn)
            wait_gathers(k)
            compute(k)

            @pl.when(j >= 1)
            def _():
                drain_scatters(k2)
            issue_scatters(k)
        return carry

    lax.fori_loop(0, CHUNKS // 3 - 1, superstep, 0)
    for k in range(3):
        j = CHUNKS - 3 + k
        kn = (k + 1) % 3
        k2 = (k + 2) % 3
        if j + 2 < CHUNKS:
            issue_idx(j + 2, k2)
        if j + 1 < CHUNKS:
            wait_idx(j + 1, kn)
            issue_gathers(kn)
        wait_gathers(k)
        compute(k)
        drain_scatters(k2)
        issue_scatters(k)
    drain_scatters((CHUNKS - 1) % 3)

    plsc.subcore_barrier()

    def finalize_block(row0, nrows):
        pltpu.sync_copy(acc_sh.at[pl.ds(row0, nrows)], xr_v[0].at[pl.ds(0, nrows)])
        pltpu.sync_copy(ssum_sh.at[pl.ds(row0, nrows)], ex_v[0].at[pl.ds(0, nrows)])

        @plsc.parallel_loop(0, nrows, unroll=4)
        def row_body(b):
            s_row = ex_v[0][b, :] + 1e-16
            for h in range(H):
                d = s_row[h]
                sl = pl.ds(h * DH, DH)
                xr_v[0][b, sl] = jnp.maximum(xr_v[0][b, sl] / d, 0.0)

        pltpu.sync_copy(xr_v[0].at[pl.ds(0, nrows)], acc_out.at[pl.ds(row0, nrows)])

    full, rem = divmod(STRIPE, B)
    for t in range(full):
        finalize_block(sid * STRIPE + t * B, B)
    if rem:
        finalize_block(sid * STRIPE + full * B, rem)


def _sc_body(A_eo, Bd_eo, X_eo, src_eo, dst_eo, cc_eo,
             A_oo, Bd_oo, X_oo, src_oo, dst_oo, cc_oo,
             acc_eo, acc_oo,
             src_v, dst_v, asrc_v, adst_v, ex_v, xr_v, cc_v,
             acc_sh, ssum_sh, sem_i, sem_g, sem_s):
    cid = lax.axis_index("c")
    sid = lax.axis_index("s")

    z = jnp.zeros((16,), jnp.float32)

    @plsc.parallel_loop(0, B, unroll=4)
    def zero_body(b):
        for h in range(H):
            xr_v[0][b, pl.ds(h * DH, DH)] = z
        ex_v[0][b, :] = z
    full, rem = divmod(STRIPE, B)
    for k in range(full):
        rows = pl.ds(sid * STRIPE + k * B, B)
        pltpu.sync_copy(xr_v[0], acc_sh.at[rows])
        pltpu.sync_copy(ex_v[0], ssum_sh.at[rows])
    if rem:
        rows = pl.ds(sid * STRIPE + full * B, rem)
        pltpu.sync_copy(xr_v[0].at[pl.ds(0, rem)], acc_sh.at[rows])
        pltpu.sync_copy(ex_v[0].at[pl.ds(0, rem)], ssum_sh.at[rows])
    plsc.subcore_barrier()

    @pl.when(cid == 0)
    def _():
        _edge_pass(sid, A_eo, Bd_eo, X_eo, src_eo, dst_eo, cc_eo,
                   acc_eo,
                   src_v, dst_v, asrc_v, adst_v, ex_v, xr_v, cc_v,
                   acc_sh, ssum_sh, sem_i, sem_g, sem_s)

    @pl.when(cid == 1)
    def _():
        _edge_pass(sid, A_oo, Bd_oo, X_oo, src_oo, dst_oo, cc_oo,
                   acc_oo,
                   src_v, dst_v, asrc_v, adst_v, ex_v, xr_v, cc_v,
                   acc_sh, ssum_sh, sem_i, sem_g, sem_s)


@jax.jit
def _sc_call(A_eo, Bd_eo, X_eo, src_eo, dst_eo, cc_eo,
             A_oo, Bd_oo, X_oo, src_oo, dst_oo, cc_oo):
    mesh = plsc.VectorSubcoreMesh(core_axis_name="c", subcore_axis_name="s")
    f = pl.kernel(
        _sc_body,
        mesh=mesh,
        compiler_params=pltpu.CompilerParams(use_tc_tiling_on_sc=False),
        out_type=(
            jax.ShapeDtypeStruct((NP, C), jnp.float32),
            jax.ShapeDtypeStruct((NP, C), jnp.float32),
        ),
        scratch_types=[
            [pltpu.VMEM((B,), jnp.int32)] * 3,
            [pltpu.VMEM((B,), jnp.int32)] * 3,
            [pltpu.VMEM((B, 16), jnp.float32)] * 3,
            [pltpu.VMEM((B, 16), jnp.float32)] * 3,
            [pltpu.VMEM((B, 16), jnp.float32)] * 3,
            [pltpu.VMEM((B, C), jnp.float32)] * 3,
            pltpu.VMEM((16,), jnp.float32),
            pltpu.VMEM_SHARED((NP, C), jnp.float32),
            pltpu.VMEM_SHARED((NP, 16), jnp.float32),
            [pltpu.SemaphoreType.DMA] * 3,
            [pltpu.SemaphoreType.DMA] * 3,
            [pltpu.SemaphoreType.DMA] * 3,
        ],
    )
    return f(A_eo, Bd_eo, X_eo, src_eo, dst_eo, cc_eo,
             A_oo, Bd_oo, X_oo, src_oo, dst_oo, cc_oo)


BLK = 400


def _post_body(eo_ref, oo_ref, kW_ref, kb_ref, q_ref, lW_ref, lb_ref,
               out_ref, acc_ref):
    i = pl.program_id(0)

    @pl.when(i == 0)
    def _():
        acc_ref[...] = jnp.zeros_like(acc_ref)

    eo = eo_ref[...]
    oo = oo_ref[...]
    kW = kW_ref[...]
    t_eo = jnp.tanh(jnp.dot(eo, kW, preferred_element_type=jnp.float32)
                    + kb_ref[...])
    t_oo = jnp.tanh(jnp.dot(oo, kW, preferred_element_type=jnp.float32)
                    + kb_ref[...])
    upd = jnp.stack([t_eo.sum(0), t_oo.sum(0), eo.sum(0), oo.sum(0)], axis=0)
    acc_ref[...] = acc_ref[...] + upd

    @pl.when(i == pl.num_programs(0) - 1)
    def _():
        acc = acc_ref[...] * (1.0 / N)
        kk = acc[0:2, :]
        score = jnp.sum(q_ref[...] * kk, axis=-1)
        m = jnp.max(score)
        w = jnp.exp(score - m)
        attn = w / jnp.sum(w)
        pooled = attn[0] * acc[2:3, :] + attn[1] * acc[3:4, :]
        out_ref[...] = (jnp.dot(pooled, lW_ref[...],
                                preferred_element_type=jnp.float32)
                        + lb_ref[...])


@jax.jit
def _post_call(out_eo, out_oo, kW, kb, q, lWp, lbp):
    grid = (N // BLK,)
    return pl.pallas_call(
        _post_body,
        grid=grid,
        in_specs=[
            pl.BlockSpec((BLK, C), lambda i: (i, 0)),
            pl.BlockSpec((BLK, C), lambda i: (i, 0)),
            pl.BlockSpec((C, C), lambda i: (0, 0)),
            pl.BlockSpec((C,), lambda i: (0,)),
            pl.BlockSpec((1, C), lambda i: (0, 0)),
            pl.BlockSpec((C, C), lambda i: (0, 0)),
            pl.BlockSpec((C,), lambda i: (0,)),
        ],
        out_specs=pl.BlockSpec((1, C), lambda i: (0, 0)),
        out_shape=jax.ShapeDtypeStruct((1, C), jnp.float32),
        scratch_shapes=[pltpu.VMEM((4, C), jnp.float32)],
    )(out_eo, out_oo, kW, kb, q, lWp, lbp)


def _prep_type(x_src, asrc, adst, ei):
    c = asrc.max(0) + adst.max(0)
    cc = jnp.concatenate([c, jnp.zeros((8,), jnp.float32)])
    A = jnp.pad(asrc, ((0, 0), (0, 16 - H)))
    Bd = jnp.pad(adst, ((0, 1), (0, 16 - H)), constant_values=0.0)
    Bd = Bd.at[N, :].set(-1e30)
    src = jnp.pad(ei[0].astype(jnp.int32), (0, EP - E))
    dst = jnp.pad(ei[1].astype(jnp.int32), (0, EP - E), constant_values=N)
    return A, Bd, x_src, src.reshape(EP // B, B), dst.reshape(EP // B, B), cc


def kernel(x_openie, x_entity, ei_oe, ei_eo, ei_oo, Wp_o, bp_o, Wp_e, bp_e,
           att_src_oe, att_dst_oe, att_src_eo, att_dst_eo, att_src_oo, att_dst_oo,
           k_lin_W, k_lin_b, q, lin_W, lin_b):
    x_o = x_openie @ Wp_o + bp_o
    x_e = x_entity @ Wp_e + bp_e

    def head_dot(x, att):
        return (x.reshape(N, H, DH) * att).sum(-1)

    asrc_eo = head_dot(x_e, att_src_eo)
    adst_eo = head_dot(x_o, att_dst_eo)
    asrc_oo = head_dot(x_o, att_src_oo)
    adst_oo = head_dot(x_o, att_dst_oo)

    args_eo = _prep_type(x_e, asrc_eo, adst_eo, ei_eo)
    args_oo = _prep_type(x_o, asrc_oo, adst_oo, ei_oo)

    out_eo_p, out_oo_p = _sc_call(*args_eo, *args_oo)

    lin_Wp = jnp.zeros((C, C), jnp.float32).at[:, :2].set(lin_W)
    lin_bp = jnp.zeros((C,), jnp.float32).at[:2].set(lin_b)
    res = _post_call(out_eo_p[:N], out_oo_p[:N], k_lin_W, k_lin_b, q,
                     lin_Wp, lin_bp)
    return res[:, :2]
